# baseline (speedup 1.0000x reference)
import functools

import jax
import jax.numpy as jnp
from jax import lax
from jax.experimental import pallas as pl
from jax.experimental.pallas import tpu as pltpu
from jax.experimental.pallas import tpu_sc as plsc

N = 10000
E = 320000
D = 128
NP = 10240
NC = 2
NS = 16
NW = NC * NS
E2 = 327680
EPT = E2 // NW
ER = E2 // 128
EPR = EPT // 128
NPT = NP // NS

_mesh = plsc.VectorSubcoreMesh(
    core_axis_name="c", subcore_axis_name="s", num_cores=NC, num_subcores=NS
)

_SPLAT_DNUMS = lax.GatherDimensionNumbers(
    offset_dims=(), collapsed_slice_dims=(0,), start_index_map=(0,)
)


def _lane_splat(vec, lane):
    idx = jnp.full((16, 1), lane, jnp.int32)
    return lax.gather(vec, idx, _SPLAT_DNUMS, (1,),
                      mode=lax.GatherScatterMode.PROMISE_IN_BOUNDS)


_A_CH = 16

def _deg_body(row, col, w, deg_o, wm_o, rbuf, cbuf, wbuf, zbuf, deg_s):
    cid = lax.axis_index("c")
    sid = lax.axis_index("s")
    wid = cid * NS + sid

    def _z(j, _):
        zbuf[pl.ds(j * 16, 16)] = jnp.zeros((16,), jnp.float32)
        return 0
    lax.fori_loop(0, NPT // 16, _z, 0)
    pltpu.sync_copy(zbuf, deg_s.at[pl.ds(sid * NPT, NPT)])
    plsc.subcore_barrier()

    base = wid * EPR
    for k in range(EPR // _A_CH):
        off = base + k * _A_CH
        pltpu.sync_copy(row.at[pl.ds(off, _A_CH), :], rbuf)
        pltpu.sync_copy(col.at[pl.ds(off, _A_CH), :], cbuf)
        pltpu.sync_copy(w.at[pl.ds(off, _A_CH), :], wbuf)

        def _mask(i, _):
            for q in range(8):
                sl = pl.ds(q * 16, 16)
                r = rbuf[i, sl]
                c = cbuf[i, sl]
                wv = wbuf[i, sl]
                wbuf[i, sl] = jnp.where(r == c, 0.0, wv)
            return 0
        lax.fori_loop(0, _A_CH, _mask, 0)

        pltpu.sync_copy(wbuf, wm_o.at[pl.ds(off, _A_CH), :])
        for i in range(_A_CH):
            pltpu.sync_copy(wbuf.at[i], deg_s.at[rbuf.at[i]], add=True)

    plsc.subcore_barrier()
    pltpu.sync_copy(deg_s.at[pl.ds(sid * NPT, NPT)],
                    deg_o.at[cid, pl.ds(sid * NPT, NPT)])


_deg_kernel = functools.partial(
    pl.kernel,
    out_type=(
        jax.ShapeDtypeStruct((NC, NP), jnp.float32),
        jax.ShapeDtypeStruct((ER, 128), jnp.float32),
    ),
    mesh=_mesh,
    scratch_types=[
        pltpu.VMEM((_A_CH, 128), jnp.int32),
        pltpu.VMEM((_A_CH, 128), jnp.int32),
        pltpu.VMEM((_A_CH, 128), jnp.float32),
        pltpu.VMEM((NPT,), jnp.float32),
        pltpu.VMEM_SHARED((NP,), jnp.float32),
    ],
)(_deg_body)


_C_SPLIT = 2432
_N0 = _C_SPLIT // NS
_N1 = (ER - _C_SPLIT) // NS


def _spmm_body(row, col, wm, hs, p_o,
               rbuf, cbuf, wbuf, cidx, rows, zrow, p_s, sg, st, ss0, ss1):
    cid = lax.axis_index("c")
    sid = lax.axis_index("s")

    nsl = jnp.where(cid == 0, _N0, _N1)
    base = jnp.where(cid == 0, sid * _N0, _C_SPLIT + sid * _N1)

    def _z(j, _):
        r = j // 8
        q = j - r * 8
        zrow[r, pl.ds(q * 16, 16)] = jnp.zeros((16,), jnp.float32)
        return 0
    lax.fori_loop(0, 16 * 8, _z, 0)
    for t in range(NPT // 16):
        pltpu.sync_copy(zrow, p_s.at[pl.ds(sid * NPT + t * 16, 16), :])
    plsc.subcore_barrier()

    def _stage(g):
        slot = (g % 2) * 8
        src = pl.ds(base + g * 8, 8)
        dst = pl.ds(slot, 8)
        pltpu.async_copy(row.at[src, :], rbuf.at[dst, :], st)
        pltpu.async_copy(col.at[src, :], cbuf.at[dst, :], st)
        pltpu.async_copy(wm.at[src, :], wbuf.at[dst, :], st)

    def _stage_wait():
        d = pl.ds(0, 8)
        pltpu.make_async_copy(row.at[d, :], rbuf.at[d, :], st).wait()
        pltpu.make_async_copy(col.at[d, :], cbuf.at[d, :], st).wait()
        pltpu.make_async_copy(wm.at[d, :], wbuf.at[d, :], st).wait()

    def _fire_gather(s, slot):
        rb = ((s // 8) % 2) * 8 + s % 8
        for h in range(4):
            pltpu.async_copy(hs.at[rbuf.at[rb, pl.ds(32 * h, 32)]],
                             rows.at[slot, pl.ds(32 * h, 32), :], sg)

    def _wait_gather(slot):
        for h in range(4):
            pltpu.make_async_copy(hs.at[rbuf.at[0, pl.ds(32 * h, 32)]],
                                  rows.at[slot, pl.ds(32 * h, 32), :],
                                  sg).wait()

    def _fire_scatter(slot, sem):
        for h in range(2):
            pltpu.async_copy(rows.at[slot, pl.ds(64 * h, 64), :],
                             p_s.at[cidx.at[2 * slot + h]], sem, add=True)

    def _wait_scatter(slot, sem):
        for h in range(2):
            pltpu.make_async_copy(rows.at[slot, pl.ds(64 * h, 64), :],
                                  p_s.at[cidx.at[2 * slot + h]], sem).wait()

    _stage(0)
    _stage_wait()
    _fire_gather(0, 0)

    def _half(s, par, ssp, sso):
        @pl.when(jnp.logical_and(s % 8 == 0, s + 8 < nsl))
        def _():
            _stage(s // 8 + 1)

        _wait_gather(par)

        @pl.when(s + 1 < nsl)
        def _():
            @pl.when((s + 1) % 8 == 0)
            def _():
                _stage_wait()

            @pl.when(s >= 1)
            def _():
                _wait_scatter(1 - par, sso)

            _fire_gather(s + 1, 1 - par)

        rb = ((s // 8) % 2) * 8 + s % 8

        def _scale(g, _):
            wv = wbuf[rb, pl.ds(g * 16, 16)]
            for e16 in range(16):
                nv = _lane_splat(wv, e16)
                e = g * 16 + e16
                for q in range(8):
                    sl = pl.ds(q * 16, 16)
                    rows[par, e, sl] = rows[par, e, sl] * nv
            return 0
        lax.fori_loop(0, 8, _scale, 0)

        for h in range(2):
            for q in range(4):
                sl = pl.ds(q * 16, 16)
                sr = pl.ds(64 * h + q * 16, 16)
                cidx[2 * par + h, sl] = cbuf[rb, sr]
        _fire_scatter(par, ssp)

    def _pair(p, _):
        _half(2 * p, 0, ss0, ss1)
        _half(2 * p + 1, 1, ss1, ss0)
        return 0

    lax.fori_loop(0, nsl // 2, _pair, 0)
    _wait_scatter(0, ss0)
    _wait_scatter(1, ss1)

    plsc.subcore_barrier()
    pltpu.sync_copy(p_s.at[pl.ds(sid * NPT, NPT), :],
                    p_o.at[cid, pl.ds(sid * NPT, NPT), :])


_spmm_kernel = functools.partial(
    pl.kernel,
    out_type=jax.ShapeDtypeStruct((NC, NP, D), jnp.float32),
    mesh=_mesh,
    scratch_types=[
        pltpu.VMEM((16, 128), jnp.int32),
        pltpu.VMEM((16, 128), jnp.int32),
        pltpu.VMEM((16, 128), jnp.float32),
        pltpu.VMEM((4, 64), jnp.int32),
        pltpu.VMEM((2, 128, D), jnp.float32),
        pltpu.VMEM((16, D), jnp.float32),
        pltpu.VMEM_SHARED((NP, D), jnp.float32),
        pltpu.SemaphoreType.DMA,
        pltpu.SemaphoreType.DMA,
        pltpu.SemaphoreType.DMA,
        pltpu.SemaphoreType.DMA,
    ],
)(_spmm_body)


def _dinv_body(deg_ref, out_ref):
    d = deg_ref[0] + deg_ref[1]
    out_ref[...] = jnp.where(d > 0, lax.rsqrt(jnp.where(d > 0, d, 1.0)), 0.0)


def _hs_body(dv_ref, h_ref, o_ref):
    o_ref[...] = -dv_ref[...] * h_ref[...]


def _gates_body(x_ref, h_ref, c_ref, p0_ref, p1_ref, dv_ref,
                wx_ref, t0_ref, t1_ref, cb_ref, gb_ref, o_ref):
    p = (p0_ref[...] + p1_ref[...]) * dv_ref[...]
    z = (jnp.dot(x_ref[...], wx_ref[...], preferred_element_type=jnp.float32)
         + jnp.dot(h_ref[...], t0_ref[...], preferred_element_type=jnp.float32)
         + jnp.dot(p, t1_ref[...], preferred_element_type=jnp.float32)
         + cb_ref[...] + gb_ref[...])
    gi = jax.nn.sigmoid(z[:, 0:128])
    gf = jax.nn.sigmoid(z[:, 128:256])
    gt = jnp.tanh(z[:, 256:384])
    go = jax.nn.sigmoid(z[:, 384:512])
    cn = gf * c_ref[...] + gi * gt
    o_ref[...] = go * jnp.tanh(cn)


_BLK = 1000


def kernel(X, edge_index, edge_weight, H, C,
           W_i, b_i, conv_i_W, conv_i_b,
           W_f, b_f, conv_f_W, conv_f_b,
           W_c, b_c, conv_c_W, conv_c_b,
           W_o, b_o, conv_o_W, conv_o_b):
    f32 = jnp.float32
    row = edge_index[0].astype(jnp.int32)
    col = edge_index[1].astype(jnp.int32)
    w = edge_weight.astype(f32)
    row2 = jnp.pad(row, (0, E2 - E)).reshape(ER, 128)
    col2 = jnp.pad(col, (0, E2 - E)).reshape(ER, 128)
    w2 = jnp.pad(w, (0, E2 - E)).reshape(ER, 128)

    deg2, wm2 = _deg_kernel(row2, col2, w2)

    dinv2 = pl.pallas_call(
        _dinv_body,
        out_shape=jax.ShapeDtypeStruct((NP // 128, 128), f32),
    )(deg2.reshape(NC, NP // 128, 128))
    dinv_col = dinv2.reshape(NP, 1)[:N]

    hs = pl.pallas_call(
        _hs_body,
        grid=(N // _BLK,),
        in_specs=[
            pl.BlockSpec((_BLK, 1), lambda i: (i, 0)),
            pl.BlockSpec((_BLK, D), lambda i: (i, 0)),
        ],
        out_specs=pl.BlockSpec((_BLK, D), lambda i: (i, 0)),
        out_shape=jax.ShapeDtypeStruct((N, D), f32),
    )(dinv_col, H)

    p2 = _spmm_kernel(row2, col2, wm2, hs)

    wx = jnp.concatenate([W_i, W_f, W_c, W_o], axis=1)
    t0 = jnp.concatenate(
        [conv_i_W[0], conv_f_W[0], conv_c_W[0], conv_o_W[0]], axis=1)
    t1 = jnp.concatenate(
        [conv_i_W[1], conv_f_W[1], conv_c_W[1], conv_o_W[1]], axis=1)
    cb = jnp.concatenate(
        [conv_i_b, conv_f_b, conv_c_b, conv_o_b]).reshape(1, 4 * D)
    gb = jnp.concatenate([b_i, b_f, b_c, b_o], axis=1)

    full = pl.BlockSpec((128, 4 * D), lambda i: (0, 0))
    brow = pl.BlockSpec((1, 4 * D), lambda i: (0, 0))
    nblk = pl.BlockSpec((_BLK, D), lambda i: (i, 0))
    h_new = pl.pallas_call(
        _gates_body,
        grid=(N // _BLK,),
        in_specs=[nblk, nblk, nblk, nblk, nblk,
                  pl.BlockSpec((_BLK, 1), lambda i: (i, 0)),
                  full, full, full, brow, brow],
        out_specs=nblk,
        out_shape=jax.ShapeDtypeStruct((N, D), f32),
    )(X, H, C, p2[0, :N], p2[1, :N], dinv_col, wx, t0, t1, cb, gb)
    return h_new


# speedup vs baseline: 1.0021x; 1.0021x over previous
import functools

import jax
import jax.numpy as jnp
from jax import lax
from jax.experimental import pallas as pl
from jax.experimental.pallas import tpu as pltpu
from jax.experimental.pallas import tpu_sc as plsc

N = 10000
E = 320000
D = 128
NP = 10240
NC = 2
NS = 16
NW = NC * NS
E2 = 327680
EPT = E2 // NW
ER = E2 // 128
EPR = EPT // 128
NPT = NP // NS

_mesh = plsc.VectorSubcoreMesh(
    core_axis_name="c", subcore_axis_name="s", num_cores=NC, num_subcores=NS
)

_SPLAT_DNUMS = lax.GatherDimensionNumbers(
    offset_dims=(), collapsed_slice_dims=(0,), start_index_map=(0,)
)


def _lane_splat(vec, lane):
    idx = jnp.full((16, 1), lane, jnp.int32)
    return lax.gather(vec, idx, _SPLAT_DNUMS, (1,),
                      mode=lax.GatherScatterMode.PROMISE_IN_BOUNDS)


_A_CH = 16

def _deg_body(row, col, w, deg_o, wm_o, rbuf, cbuf, wbuf, zbuf, deg_s):
    cid = lax.axis_index("c")
    sid = lax.axis_index("s")
    wid = cid * NS + sid

    def _z(j, _):
        zbuf[pl.ds(j * 16, 16)] = jnp.zeros((16,), jnp.float32)
        return 0
    lax.fori_loop(0, NPT // 16, _z, 0)
    pltpu.sync_copy(zbuf, deg_s.at[pl.ds(sid * NPT, NPT)])
    plsc.subcore_barrier()

    base = wid * EPR
    for k in range(EPR // _A_CH):
        off = base + k * _A_CH
        pltpu.sync_copy(row.at[pl.ds(off, _A_CH), :], rbuf)
        pltpu.sync_copy(col.at[pl.ds(off, _A_CH), :], cbuf)
        pltpu.sync_copy(w.at[pl.ds(off, _A_CH), :], wbuf)

        def _mask(i, _):
            for q in range(8):
                sl = pl.ds(q * 16, 16)
                r = rbuf[i, sl]
                c = cbuf[i, sl]
                wv = wbuf[i, sl]
                wbuf[i, sl] = jnp.where(r == c, 0.0, wv)
            return 0
        lax.fori_loop(0, _A_CH, _mask, 0)

        pltpu.sync_copy(wbuf, wm_o.at[pl.ds(off, _A_CH), :])
        for i in range(_A_CH):
            pltpu.sync_copy(wbuf.at[i], deg_s.at[rbuf.at[i]], add=True)

    plsc.subcore_barrier()
    pltpu.sync_copy(deg_s.at[pl.ds(sid * NPT, NPT)],
                    deg_o.at[cid, pl.ds(sid * NPT, NPT)])


_deg_kernel = functools.partial(
    pl.kernel,
    out_type=(
        jax.ShapeDtypeStruct((NC, NP), jnp.float32),
        jax.ShapeDtypeStruct((ER, 128), jnp.float32),
    ),
    mesh=_mesh,
    scratch_types=[
        pltpu.VMEM((_A_CH, 128), jnp.int32),
        pltpu.VMEM((_A_CH, 128), jnp.int32),
        pltpu.VMEM((_A_CH, 128), jnp.float32),
        pltpu.VMEM((NPT,), jnp.float32),
        pltpu.VMEM_SHARED((NP,), jnp.float32),
    ],
)(_deg_body)


_C_SPLIT = 2304
_N0 = _C_SPLIT // NS
_N1 = (ER - _C_SPLIT) // NS


def _spmm_body(row, col, wm, hs, p_o,
               rbuf, cbuf, wbuf, cidx, rows, zrow, p_s, sg, st, ss0, ss1):
    cid = lax.axis_index("c")
    sid = lax.axis_index("s")

    nsl = jnp.where(cid == 0, _N0, _N1)
    base = jnp.where(cid == 0, sid * _N0, _C_SPLIT + sid * _N1)

    def _z(j, _):
        r = j // 8
        q = j - r * 8
        zrow[r, pl.ds(q * 16, 16)] = jnp.zeros((16,), jnp.float32)
        return 0
    lax.fori_loop(0, 16 * 8, _z, 0)
    for t in range(NPT // 16):
        pltpu.sync_copy(zrow, p_s.at[pl.ds(sid * NPT + t * 16, 16), :])
    plsc.subcore_barrier()

    def _stage(g):
        slot = (g % 2) * 8
        src = pl.ds(base + g * 8, 8)
        dst = pl.ds(slot, 8)
        pltpu.async_copy(row.at[src, :], rbuf.at[dst, :], st)
        pltpu.async_copy(col.at[src, :], cbuf.at[dst, :], st)
        pltpu.async_copy(wm.at[src, :], wbuf.at[dst, :], st)

    def _stage_wait():
        d = pl.ds(0, 8)
        pltpu.make_async_copy(row.at[d, :], rbuf.at[d, :], st).wait()
        pltpu.make_async_copy(col.at[d, :], cbuf.at[d, :], st).wait()
        pltpu.make_async_copy(wm.at[d, :], wbuf.at[d, :], st).wait()

    def _fire_gather(s, slot):
        rb = ((s // 8) % 2) * 8 + s % 8
        for h in range(4):
            pltpu.async_copy(hs.at[rbuf.at[rb, pl.ds(32 * h, 32)]],
                             rows.at[slot, pl.ds(32 * h, 32), :], sg)

    def _wait_gather(slot):
        for h in range(4):
            pltpu.make_async_copy(hs.at[rbuf.at[0, pl.ds(32 * h, 32)]],
                                  rows.at[slot, pl.ds(32 * h, 32), :],
                                  sg).wait()

    def _fire_scatter(slot, sem):
        for h in range(2):
            pltpu.async_copy(rows.at[slot, pl.ds(64 * h, 64), :],
                             p_s.at[cidx.at[2 * slot + h]], sem, add=True)

    def _wait_scatter(slot, sem):
        for h in range(2):
            pltpu.make_async_copy(rows.at[slot, pl.ds(64 * h, 64), :],
                                  p_s.at[cidx.at[2 * slot + h]], sem).wait()

    _stage(0)
    _stage_wait()
    _fire_gather(0, 0)

    def _half(s, par, ssp, sso):
        @pl.when(jnp.logical_and(s % 8 == 0, s + 8 < nsl))
        def _():
            _stage(s // 8 + 1)

        _wait_gather(par)

        @pl.when(s + 1 < nsl)
        def _():
            @pl.when((s + 1) % 8 == 0)
            def _():
                _stage_wait()

            @pl.when(s >= 1)
            def _():
                _wait_scatter(1 - par, sso)

            _fire_gather(s + 1, 1 - par)

        rb = ((s // 8) % 2) * 8 + s % 8

        def _scale(g, _):
            wv = wbuf[rb, pl.ds(g * 16, 16)]
            for e16 in range(16):
                nv = _lane_splat(wv, e16)
                e = g * 16 + e16
                for q in range(8):
                    sl = pl.ds(q * 16, 16)
                    rows[par, e, sl] = rows[par, e, sl] * nv
            return 0
        lax.fori_loop(0, 8, _scale, 0)

        for h in range(2):
            for q in range(4):
                sl = pl.ds(q * 16, 16)
                sr = pl.ds(64 * h + q * 16, 16)
                cidx[2 * par + h, sl] = cbuf[rb, sr]
        _fire_scatter(par, ssp)

    def _pair(p, _):
        _half(2 * p, 0, ss0, ss1)
        _half(2 * p + 1, 1, ss1, ss0)
        return 0

    lax.fori_loop(0, nsl // 2, _pair, 0)
    _wait_scatter(0, ss0)
    _wait_scatter(1, ss1)

    plsc.subcore_barrier()
    pltpu.sync_copy(p_s.at[pl.ds(sid * NPT, NPT), :],
                    p_o.at[cid, pl.ds(sid * NPT, NPT), :])


_spmm_kernel = functools.partial(
    pl.kernel,
    out_type=jax.ShapeDtypeStruct((NC, NP, D), jnp.float32),
    mesh=_mesh,
    scratch_types=[
        pltpu.VMEM((16, 128), jnp.int32),
        pltpu.VMEM((16, 128), jnp.int32),
        pltpu.VMEM((16, 128), jnp.float32),
        pltpu.VMEM((4, 64), jnp.int32),
        pltpu.VMEM((2, 128, D), jnp.float32),
        pltpu.VMEM((16, D), jnp.float32),
        pltpu.VMEM_SHARED((NP, D), jnp.float32),
        pltpu.SemaphoreType.DMA,
        pltpu.SemaphoreType.DMA,
        pltpu.SemaphoreType.DMA,
        pltpu.SemaphoreType.DMA,
    ],
)(_spmm_body)


def _dinv_body(deg_ref, out_ref):
    d = deg_ref[0] + deg_ref[1]
    out_ref[...] = jnp.where(d > 0, lax.rsqrt(jnp.where(d > 0, d, 1.0)), 0.0)


def _hs_body(dv_ref, h_ref, o_ref):
    o_ref[...] = -dv_ref[...] * h_ref[...]


def _gates_body(x_ref, h_ref, c_ref, p0_ref, p1_ref, dv_ref,
                wx_ref, t0_ref, t1_ref, cb_ref, gb_ref, o_ref):
    p = (p0_ref[...] + p1_ref[...]) * dv_ref[...]
    z = (jnp.dot(x_ref[...], wx_ref[...], preferred_element_type=jnp.float32)
         + jnp.dot(h_ref[...], t0_ref[...], preferred_element_type=jnp.float32)
         + jnp.dot(p, t1_ref[...], preferred_element_type=jnp.float32)
         + cb_ref[...] + gb_ref[...])
    gi = jax.nn.sigmoid(z[:, 0:128])
    gf = jax.nn.sigmoid(z[:, 128:256])
    gt = jnp.tanh(z[:, 256:384])
    go = jax.nn.sigmoid(z[:, 384:512])
    cn = gf * c_ref[...] + gi * gt
    o_ref[...] = go * jnp.tanh(cn)


_BLK = 1000


def kernel(X, edge_index, edge_weight, H, C,
           W_i, b_i, conv_i_W, conv_i_b,
           W_f, b_f, conv_f_W, conv_f_b,
           W_c, b_c, conv_c_W, conv_c_b,
           W_o, b_o, conv_o_W, conv_o_b):
    f32 = jnp.float32
    row = edge_index[0].astype(jnp.int32)
    col = edge_index[1].astype(jnp.int32)
    w = edge_weight.astype(f32)
    row2 = jnp.pad(row, (0, E2 - E)).reshape(ER, 128)
    col2 = jnp.pad(col, (0, E2 - E)).reshape(ER, 128)
    w2 = jnp.pad(w, (0, E2 - E)).reshape(ER, 128)

    deg2, wm2 = _deg_kernel(row2, col2, w2)

    dinv2 = pl.pallas_call(
        _dinv_body,
        out_shape=jax.ShapeDtypeStruct((NP // 128, 128), f32),
    )(deg2.reshape(NC, NP // 128, 128))
    dinv_col = dinv2.reshape(NP, 1)[:N]

    hs = pl.pallas_call(
        _hs_body,
        grid=(N // _BLK,),
        in_specs=[
            pl.BlockSpec((_BLK, 1), lambda i: (i, 0)),
            pl.BlockSpec((_BLK, D), lambda i: (i, 0)),
        ],
        out_specs=pl.BlockSpec((_BLK, D), lambda i: (i, 0)),
        out_shape=jax.ShapeDtypeStruct((N, D), f32),
    )(dinv_col, H)

    p2 = _spmm_kernel(row2, col2, wm2, hs)

    wx = jnp.concatenate([W_i, W_f, W_c, W_o], axis=1)
    t0 = jnp.concatenate(
        [conv_i_W[0], conv_f_W[0], conv_c_W[0], conv_o_W[0]], axis=1)
    t1 = jnp.concatenate(
        [conv_i_W[1], conv_f_W[1], conv_c_W[1], conv_o_W[1]], axis=1)
    cb = jnp.concatenate(
        [conv_i_b, conv_f_b, conv_c_b, conv_o_b]).reshape(1, 4 * D)
    gb = jnp.concatenate([b_i, b_f, b_c, b_o], axis=1)

    full = pl.BlockSpec((128, 4 * D), lambda i: (0, 0))
    brow = pl.BlockSpec((1, 4 * D), lambda i: (0, 0))
    nblk = pl.BlockSpec((_BLK, D), lambda i: (i, 0))
    h_new = pl.pallas_call(
        _gates_body,
        grid=(N // _BLK,),
        in_specs=[nblk, nblk, nblk, nblk, nblk,
                  pl.BlockSpec((_BLK, 1), lambda i: (i, 0)),
                  full, full, full, brow, brow],
        out_specs=nblk,
        out_shape=jax.ShapeDtypeStruct((N, D), f32),
    )(X, H, C, p2[0, :N], p2[1, :N], dinv_col, wx, t0, t1, cb, gb)
    return h_new


# speedup vs baseline: 1.0094x; 1.0073x over previous
import functools

import jax
import jax.numpy as jnp
from jax import lax
from jax.experimental import pallas as pl
from jax.experimental.pallas import tpu as pltpu
from jax.experimental.pallas import tpu_sc as plsc

N = 10000
E = 320000
D = 128
NP = 10240
NC = 2
NS = 16
NW = NC * NS
E2 = 327680
EPT = E2 // NW
ER = E2 // 128
EPR = EPT // 128
NPT = NP // NS

_mesh = plsc.VectorSubcoreMesh(
    core_axis_name="c", subcore_axis_name="s", num_cores=NC, num_subcores=NS
)

_SPLAT_DNUMS = lax.GatherDimensionNumbers(
    offset_dims=(), collapsed_slice_dims=(0,), start_index_map=(0,)
)


def _lane_splat(vec, lane):
    idx = jnp.full((16, 1), lane, jnp.int32)
    return lax.gather(vec, idx, _SPLAT_DNUMS, (1,),
                      mode=lax.GatherScatterMode.PROMISE_IN_BOUNDS)


_A_CH = 16

def _deg_body(row, col, w, deg_o, wm_o, rbuf, cbuf, wbuf, zbuf, deg_s):
    cid = lax.axis_index("c")
    sid = lax.axis_index("s")
    wid = cid * NS + sid

    def _z(j, _):
        zbuf[pl.ds(j * 16, 16)] = jnp.zeros((16,), jnp.float32)
        return 0
    lax.fori_loop(0, NPT // 16, _z, 0)
    pltpu.sync_copy(zbuf, deg_s.at[pl.ds(sid * NPT, NPT)])
    plsc.subcore_barrier()

    base = wid * EPR
    for k in range(EPR // _A_CH):
        off = base + k * _A_CH
        pltpu.sync_copy(row.at[pl.ds(off, _A_CH), :], rbuf)
        pltpu.sync_copy(col.at[pl.ds(off, _A_CH), :], cbuf)
        pltpu.sync_copy(w.at[pl.ds(off, _A_CH), :], wbuf)

        def _mask(i, _):
            for q in range(8):
                sl = pl.ds(q * 16, 16)
                r = rbuf[i, sl]
                c = cbuf[i, sl]
                wv = wbuf[i, sl]
                wbuf[i, sl] = jnp.where(r == c, 0.0, wv)
            return 0
        lax.fori_loop(0, _A_CH, _mask, 0)

        pltpu.sync_copy(wbuf, wm_o.at[pl.ds(off, _A_CH), :])
        for i in range(_A_CH):
            pltpu.sync_copy(wbuf.at[i], deg_s.at[rbuf.at[i]], add=True)

    plsc.subcore_barrier()
    pltpu.sync_copy(deg_s.at[pl.ds(sid * NPT, NPT)],
                    deg_o.at[cid, pl.ds(sid * NPT, NPT)])


_deg_kernel = functools.partial(
    pl.kernel,
    out_type=(
        jax.ShapeDtypeStruct((NC, NP), jnp.float32),
        jax.ShapeDtypeStruct((ER, 128), jnp.float32),
    ),
    mesh=_mesh,
    scratch_types=[
        pltpu.VMEM((_A_CH, 128), jnp.int32),
        pltpu.VMEM((_A_CH, 128), jnp.int32),
        pltpu.VMEM((_A_CH, 128), jnp.float32),
        pltpu.VMEM((NPT,), jnp.float32),
        pltpu.VMEM_SHARED((NP,), jnp.float32),
    ],
)(_deg_body)


_C_SPLIT = 2304
_N0 = _C_SPLIT // NS
_N1 = (ER - _C_SPLIT) // NS


def _spmm_body(row, col, wm, hs, p_o,
               rbuf, cbuf, wbuf, cidx, rows, zrow, p_s, sg, st, ss0, ss1):
    cid = lax.axis_index("c")
    sid = lax.axis_index("s")

    nsl = jnp.where(cid == 0, _N0, _N1)
    base = jnp.where(cid == 0, sid * _N0, _C_SPLIT + sid * _N1)

    def _z(j, _):
        r = j // 8
        q = j - r * 8
        zrow[r, pl.ds(q * 16, 16)] = jnp.zeros((16,), jnp.float32)
        return 0
    lax.fori_loop(0, 16 * 8, _z, 0)
    for t in range(NPT // 16):
        pltpu.sync_copy(zrow, p_s.at[pl.ds(sid * NPT + t * 16, 16), :])
    plsc.subcore_barrier()

    def _stage(g):
        slot = (g % 2) * 8
        src = pl.ds(base + g * 8, 8)
        dst = pl.ds(slot, 8)
        pltpu.async_copy(row.at[src, :], rbuf.at[dst, :], st)
        pltpu.async_copy(col.at[src, :], cbuf.at[dst, :], st)
        pltpu.async_copy(wm.at[src, :], wbuf.at[dst, :], st)

    def _stage_wait():
        d = pl.ds(0, 8)
        pltpu.make_async_copy(row.at[d, :], rbuf.at[d, :], st).wait()
        pltpu.make_async_copy(col.at[d, :], cbuf.at[d, :], st).wait()
        pltpu.make_async_copy(wm.at[d, :], wbuf.at[d, :], st).wait()

    def _fire_gather(s, slot):
        rb = ((s // 8) % 2) * 8 + s % 8
        for h in range(4):
            pltpu.async_copy(hs.at[rbuf.at[rb, pl.ds(32 * h, 32)]],
                             rows.at[slot, pl.ds(32 * h, 32), :], sg)

    def _wait_gather(slot):
        for h in range(4):
            pltpu.make_async_copy(hs.at[rbuf.at[0, pl.ds(32 * h, 32)]],
                                  rows.at[slot, pl.ds(32 * h, 32), :],
                                  sg).wait()

    def _fire_scatter(slot, sem):
        for h in range(2):
            pltpu.async_copy(rows.at[slot, pl.ds(64 * h, 64), :],
                             p_s.at[cidx.at[2 * slot + h]], sem, add=True)

    def _wait_scatter(slot, sem):
        for h in range(2):
            pltpu.make_async_copy(rows.at[slot, pl.ds(64 * h, 64), :],
                                  p_s.at[cidx.at[2 * slot + h]], sem).wait()

    _stage(0)
    _stage_wait()
    _fire_gather(0, 0)

    def _half(s, par, ssp, sso):
        @pl.when(jnp.logical_and(s % 8 == 0, s + 8 < nsl))
        def _():
            _stage(s // 8 + 1)

        _wait_gather(par)

        @pl.when(s + 1 < nsl)
        def _():
            @pl.when((s + 1) % 8 == 0)
            def _():
                _stage_wait()

            @pl.when(s >= 1)
            def _():
                _wait_scatter(1 - par, sso)

            _fire_gather(s + 1, 1 - par)

        rb = ((s // 8) % 2) * 8 + s % 8

        def _scale(g, _):
            wv = wbuf[rb, pl.ds(g * 16, 16)]
            for e16 in range(16):
                nv = _lane_splat(wv, e16)
                e = g * 16 + e16
                for q in range(8):
                    sl = pl.ds(q * 16, 16)
                    rows[par, e, sl] = rows[par, e, sl] * nv
            return 0
        lax.fori_loop(0, 8, _scale, 0)

        for h in range(2):
            for q in range(4):
                sl = pl.ds(q * 16, 16)
                sr = pl.ds(64 * h + q * 16, 16)
                cidx[2 * par + h, sl] = cbuf[rb, sr]
        _fire_scatter(par, ssp)

    def _pair(p, _):
        _half(2 * p, 0, ss0, ss1)
        _half(2 * p + 1, 1, ss1, ss0)
        return 0

    lax.fori_loop(0, nsl // 2, _pair, 0)
    _wait_scatter(0, ss0)
    _wait_scatter(1, ss1)

    plsc.subcore_barrier()
    pltpu.sync_copy(p_s.at[pl.ds(sid * NPT, NPT), :],
                    p_o.at[cid, pl.ds(sid * NPT, NPT), :])


_spmm_kernel = functools.partial(
    pl.kernel,
    out_type=jax.ShapeDtypeStruct((NC, NP, D), jnp.float32),
    mesh=_mesh,
    scratch_types=[
        pltpu.VMEM((16, 128), jnp.int32),
        pltpu.VMEM((16, 128), jnp.int32),
        pltpu.VMEM((16, 128), jnp.float32),
        pltpu.VMEM((4, 64), jnp.int32),
        pltpu.VMEM((2, 128, D), jnp.float32),
        pltpu.VMEM((16, D), jnp.float32),
        pltpu.VMEM_SHARED((NP, D), jnp.float32),
        pltpu.SemaphoreType.DMA,
        pltpu.SemaphoreType.DMA,
        pltpu.SemaphoreType.DMA,
        pltpu.SemaphoreType.DMA,
    ],
)(_spmm_body)


def _dinv_body(deg_ref, out_ref):
    d = deg_ref[0] + deg_ref[1]
    out_ref[...] = jnp.where(d > 0, lax.rsqrt(jnp.where(d > 0, d, 1.0)), 0.0)


def _hs_body(dv_ref, h_ref, o_ref):
    o_ref[...] = -dv_ref[...] * h_ref[...]


def _gates_body(x_ref, h_ref, c_ref, p0_ref, p1_ref, dv_ref,
                wx_ref, t0_ref, t1_ref, cb_ref, gb_ref, o_ref):
    p = (p0_ref[...] + p1_ref[...]) * dv_ref[...]
    z = (jnp.dot(x_ref[...], wx_ref[...], preferred_element_type=jnp.float32)
         + jnp.dot(h_ref[...], t0_ref[...], preferred_element_type=jnp.float32)
         + jnp.dot(p, t1_ref[...], preferred_element_type=jnp.float32)
         + cb_ref[...] + gb_ref[...])
    gi = jax.nn.sigmoid(z[:, 0:128])
    gf = jax.nn.sigmoid(z[:, 128:256])
    gt = jnp.tanh(z[:, 256:384])
    go = jax.nn.sigmoid(z[:, 384:512])
    cn = gf * c_ref[...] + gi * gt
    o_ref[...] = go * jnp.tanh(cn)


_BLK = 2000


def kernel(X, edge_index, edge_weight, H, C,
           W_i, b_i, conv_i_W, conv_i_b,
           W_f, b_f, conv_f_W, conv_f_b,
           W_c, b_c, conv_c_W, conv_c_b,
           W_o, b_o, conv_o_W, conv_o_b):
    f32 = jnp.float32
    row = edge_index[0].astype(jnp.int32)
    col = edge_index[1].astype(jnp.int32)
    w = edge_weight.astype(f32)
    row2 = jnp.pad(row, (0, E2 - E)).reshape(ER, 128)
    col2 = jnp.pad(col, (0, E2 - E)).reshape(ER, 128)
    w2 = jnp.pad(w, (0, E2 - E)).reshape(ER, 128)

    deg2, wm2 = _deg_kernel(row2, col2, w2)

    dinv2 = pl.pallas_call(
        _dinv_body,
        out_shape=jax.ShapeDtypeStruct((NP // 128, 128), f32),
    )(deg2.reshape(NC, NP // 128, 128))
    dinv_col = dinv2.reshape(NP, 1)[:N]

    hs = pl.pallas_call(
        _hs_body,
        grid=(N // _BLK,),
        in_specs=[
            pl.BlockSpec((_BLK, 1), lambda i: (i, 0)),
            pl.BlockSpec((_BLK, D), lambda i: (i, 0)),
        ],
        out_specs=pl.BlockSpec((_BLK, D), lambda i: (i, 0)),
        out_shape=jax.ShapeDtypeStruct((N, D), f32),
    )(dinv_col, H)

    p2 = _spmm_kernel(row2, col2, wm2, hs)

    wx = jnp.concatenate([W_i, W_f, W_c, W_o], axis=1)
    t0 = jnp.concatenate(
        [conv_i_W[0], conv_f_W[0], conv_c_W[0], conv_o_W[0]], axis=1)
    t1 = jnp.concatenate(
        [conv_i_W[1], conv_f_W[1], conv_c_W[1], conv_o_W[1]], axis=1)
    cb = jnp.concatenate(
        [conv_i_b, conv_f_b, conv_c_b, conv_o_b]).reshape(1, 4 * D)
    gb = jnp.concatenate([b_i, b_f, b_c, b_o], axis=1)

    full = pl.BlockSpec((128, 4 * D), lambda i: (0, 0))
    brow = pl.BlockSpec((1, 4 * D), lambda i: (0, 0))
    nblk = pl.BlockSpec((_BLK, D), lambda i: (i, 0))
    h_new = pl.pallas_call(
        _gates_body,
        grid=(N // _BLK,),
        in_specs=[nblk, nblk, nblk, nblk, nblk,
                  pl.BlockSpec((_BLK, 1), lambda i: (i, 0)),
                  full, full, full, brow, brow],
        out_specs=nblk,
        out_shape=jax.ShapeDtypeStruct((N, D), f32),
    )(X, H, C, p2[0, :N], p2[1, :N], dinv_col, wx, t0, t1, cb, gb)
    return h_new
